# baseline (speedup 1.0000x reference)
import functools

import jax
import jax.numpy as jnp
from jax import lax
from jax.experimental import pallas as pl
from jax.experimental.pallas import tpu as pltpu
from jax.experimental.pallas import tpu_sc as plsc

_D = 64
_W = _D // 2
_GATHER = 128
_NBUF = 4
_OBUF = 2
_L = 16
_TCBLK = 1600


@functools.lru_cache(maxsize=None)
def _build_sc(batch, n_rows):
    info = plsc.get_sparse_core_info()
    nc, ns = info.num_cores, info.num_subcores
    nw = nc * ns
    assert batch % (nw * _GATHER) == 0
    b_per_w = batch // nw
    n_blocks = b_per_w // _GATHER

    mesh = plsc.VectorSubcoreMesh(core_axis_name="c", subcore_axis_name="s")
    scratch = (
        [pltpu.VMEM((b_per_w,), jnp.int32)]
        + [pltpu.VMEM((_NBUF, _GATHER, _W), jnp.int32)]
        + [pltpu.VMEM((_OBUF, _GATHER // 2, _D), jnp.int32)]
        + [pltpu.SemaphoreType.DMA] * (_NBUF + _OBUF)
    )

    @functools.partial(
        pl.kernel,
        mesh=mesh,
        out_type=jax.ShapeDtypeStruct((batch // 2, _D), jnp.int32),
        scratch_types=scratch,
        compiler_params=pltpu.CompilerParams(
            use_tc_tiling_on_sc=False, needs_layout_passes=False),
    )
    def emb(idx_hbm, table32, z_hbm, idx_v, rows_v, obuf_v, *sems):
        gsem = sems[:_NBUF]
        wsem = sems[_NBUF:]

        wid = lax.axis_index("s") * nc + lax.axis_index("c")
        base = wid * b_per_w
        pltpu.sync_copy(idx_hbm.at[pl.ds(base, b_per_w)], idx_v)

        def fire_gather(g, b):
            pltpu.async_copy(
                table32.at[idx_v.at[pl.ds(pl.multiple_of(g * _GATHER, _GATHER), _GATHER)]],
                rows_v.at[b], gsem[b])

        def wait_gather(g, b):
            pltpu.make_async_copy(
                table32.at[idx_v.at[pl.ds(pl.multiple_of(g * _GATHER, _GATHER), _GATHER)]],
                rows_v.at[b], gsem[b]).wait()

        def fire_write(g, o):
            pltpu.async_copy(
                obuf_v.at[o],
                z_hbm.at[pl.ds(pl.multiple_of((base + g * _GATHER) // 2, _GATHER // 2), _GATHER // 2)],
                wsem[o])

        def wait_write(g, o):
            pltpu.make_async_copy(
                obuf_v.at[o],
                z_hbm.at[pl.ds(pl.multiple_of((base + g * _GATHER) // 2, _GATHER // 2), _GATHER // 2)],
                wsem[o]).wait()

        for b in range(_NBUF):
            fire_gather(b, b)

        iota = lax.iota(jnp.int32, _L)

        def block(g, b, o, first):
            wait_gather(g, b)
            if not first:
                wait_write(g - _OBUF, o)

            def pair(m, carry):
                row = 2 * m
                for t in range(_W // _L):
                    sl = pl.ds(t * _L, _L)
                    ge = rows_v[b, row, sl]
                    go = rows_v[b, row + 1, sl]
                    w_even = (ge & 0xFFFF) | (go << 16)
                    w_odd = ((ge >> 16) & 0xFFFF) | (go & jnp.int32(-65536))
                    cols = 2 * iota + (2 * _L * t)
                    mrow = jnp.full((_L,), m, jnp.int32)
                    plsc.store_scatter(obuf_v.at[o], [mrow, cols], w_even)
                    plsc.store_scatter(obuf_v.at[o], [mrow, cols + 1], w_odd)
                return carry

            lax.fori_loop(0, _GATHER // 2, pair, 0, unroll=2)
            fire_write(g, o)

            if isinstance(g, int):
                if g + _NBUF < n_blocks:
                    fire_gather(g + _NBUF, b)
            else:
                @pl.when(g + _NBUF < n_blocks)
                def _():
                    fire_gather(g + _NBUF, b)

        def group(gi, carry):
            for j in range(_NBUF):
                block(gi * _NBUF + j, j, j % _OBUF, False)
            return carry

        for j in range(_NBUF):
            block(j, j, j % _OBUF, j < _OBUF)
        lax.fori_loop(1, n_blocks // _NBUF, group, 0)
        for g in range(n_blocks - _OBUF, n_blocks):
            wait_write(g, g % _OBUF)

    return emb


def _unpack_body(hist, z_ref, o_ref):
    v = pltpu.bitcast(z_ref[...], jnp.bfloat16)
    for k in range(2 * _TCBLK // hist):
        o_ref[k] = v[k * hist:(k + 1) * hist]


@functools.lru_cache(maxsize=None)
def _build_tc(bsz, hist):
    slabs = 2 * _TCBLK // hist
    assert 2 * _TCBLK % hist == 0 and bsz % slabs == 0
    grid = bsz // slabs
    return pl.pallas_call(
        functools.partial(_unpack_body, hist),
        grid=(grid,),
        in_specs=[pl.BlockSpec((_TCBLK, _D), lambda i: (i, 0))],
        out_specs=pl.BlockSpec((slabs, hist, _D), lambda i: (i, 0, 0)),
        out_shape=jax.ShapeDtypeStruct((bsz, hist, _D), jnp.bfloat16),
        compiler_params=pltpu.CompilerParams(
            dimension_semantics=("arbitrary",)),
    )


def kernel(input_ids, weight):
    bsz, hist = input_ids.shape
    flat = input_ids.reshape(-1)
    n_rows, d = weight.shape
    w32 = jax.lax.bitcast_convert_type(
        weight.reshape(-1, 2), jnp.int32).reshape(n_rows, d // 2)
    z = _build_sc(flat.shape[0], n_rows)(flat, w32)
    return _build_tc(bsz, hist)(z)


# speedup vs baseline: 9.3631x; 9.3631x over previous
import functools

import jax
import jax.numpy as jnp
from jax import lax
from jax.experimental import pallas as pl
from jax.experimental.pallas import tpu as pltpu
from jax.experimental.pallas import tpu_sc as plsc

_D = 64
_W = _D // 2
_GATHER = 128
_NBUF = 4
_OBUF = 2
_L = 16
_TCBLK = 1600


@functools.lru_cache(maxsize=None)
def _build_sc(batch, n_rows):
    info = plsc.get_sparse_core_info()
    nc, ns = info.num_cores, info.num_subcores
    nw = nc * ns
    assert batch % (nw * _GATHER) == 0
    b_per_w = batch // nw
    n_blocks = b_per_w // _GATHER

    mesh = plsc.VectorSubcoreMesh(core_axis_name="c", subcore_axis_name="s")
    scratch = (
        [pltpu.VMEM((b_per_w,), jnp.int32)]
        + [pltpu.VMEM((_NBUF, _GATHER, _W), jnp.int32)]
        + [pltpu.VMEM((_OBUF, _GATHER // 2, _D), jnp.int32)]
        + [pltpu.SemaphoreType.DMA] * (_NBUF + _OBUF)
    )

    @functools.partial(
        pl.kernel,
        mesh=mesh,
        out_type=jax.ShapeDtypeStruct((batch // 2, _D), jnp.int32),
        scratch_types=scratch,
        compiler_params=pltpu.CompilerParams(
            use_tc_tiling_on_sc=False, needs_layout_passes=False),
    )
    def emb(idx_hbm, table32, z_hbm, idx_v, rows_v, obuf_v, *sems):
        gsem = sems[:_NBUF]
        wsem = sems[_NBUF:]

        wid = lax.axis_index("s") * nc + lax.axis_index("c")
        base = wid * b_per_w
        pltpu.sync_copy(idx_hbm.at[pl.ds(base, b_per_w)], idx_v)

        def fire_gather(g, b):
            pltpu.async_copy(
                table32.at[idx_v.at[pl.ds(pl.multiple_of(g * _GATHER, _GATHER), _GATHER)]],
                rows_v.at[b], gsem[b])

        def wait_gather(g, b):
            pltpu.make_async_copy(
                table32.at[idx_v.at[pl.ds(pl.multiple_of(g * _GATHER, _GATHER), _GATHER)]],
                rows_v.at[b], gsem[b]).wait()

        def fire_write(g, o):
            pltpu.async_copy(
                obuf_v.at[o],
                z_hbm.at[pl.ds(pl.multiple_of((base + g * _GATHER) // 2, _GATHER // 2), _GATHER // 2)],
                wsem[o])

        def wait_write(g, o):
            pltpu.make_async_copy(
                obuf_v.at[o],
                z_hbm.at[pl.ds(pl.multiple_of((base + g * _GATHER) // 2, _GATHER // 2), _GATHER // 2)],
                wsem[o]).wait()

        for b in range(_NBUF):
            fire_gather(b, b)

        iota = lax.iota(jnp.int32, _L)

        def block(g, b, o, first):
            wait_gather(g, b)
            if not first:
                wait_write(g - _OBUF, o)

            def pair(m, carry):
                row = 2 * m
                for t in range(_W // _L):
                    sl = pl.ds(t * _L, _L)
                    ge = rows_v[b, row, sl]
                    go = rows_v[b, row + 1, sl]
                    w_even = (ge & 0xFFFF) | (go << 16)
                    w_odd = ((ge >> 16) & 0xFFFF) | (go & jnp.int32(-65536))
                    cols = 2 * iota + (2 * _L * t)
                    mrow = jnp.full((_L,), m, jnp.int32)
                    plsc.store_scatter(obuf_v.at[o], [mrow, cols], w_even)
                    plsc.store_scatter(obuf_v.at[o], [mrow, cols + 1], w_odd)
                return carry

            lax.fori_loop(0, _GATHER // 2, pair, 0, unroll=2)
            fire_write(g, o)

            if isinstance(g, int):
                if g + _NBUF < n_blocks:
                    fire_gather(g + _NBUF, b)
            else:
                @pl.when(g + _NBUF < n_blocks)
                def _():
                    fire_gather(g + _NBUF, b)

        def group(gi, carry):
            for j in range(_NBUF):
                block(gi * _NBUF + j, j, j % _OBUF, False)
            return carry

        for j in range(_NBUF):
            block(j, j, j % _OBUF, j < _OBUF)
        lax.fori_loop(1, n_blocks // _NBUF, group, 0)
        for g in range(n_blocks - _OBUF, n_blocks):
            wait_write(g, g % _OBUF)

    return emb


def _unpack_body(hist, z_ref, o_ref):
    v = pltpu.bitcast(z_ref[...], jnp.bfloat16)
    for k in range(2 * _TCBLK // hist):
        o_ref[k] = v[k * hist:(k + 1) * hist]


@functools.lru_cache(maxsize=None)
def _build_tc(bsz, hist):
    slabs = 2 * _TCBLK // hist
    assert 2 * _TCBLK % hist == 0 and bsz % slabs == 0
    grid = bsz // slabs
    return pl.pallas_call(
        functools.partial(_unpack_body, hist),
        grid=(grid,),
        in_specs=[pl.BlockSpec((_TCBLK, _D), lambda i: (i, 0))],
        out_specs=pl.BlockSpec((slabs, hist, _D), lambda i: (i, 0, 0)),
        out_shape=jax.ShapeDtypeStruct((bsz, hist, _D), jnp.bfloat16),
        compiler_params=pltpu.CompilerParams(
            dimension_semantics=("arbitrary",)),
    )


def kernel(input_ids, weight):
    bsz, hist = input_ids.shape
    flat = input_ids.reshape(-1)
    n_rows, d = weight.shape
    w32 = jax.lax.bitcast_convert_type(
        weight.reshape(n_rows, d // 2, 2), jnp.int32)
    z = _build_sc(flat.shape[0], n_rows)(flat, w32)
    return _build_tc(bsz, hist)(z)
